# Initial kernel scaffold; baseline (speedup 1.0000x reference)
import jax
import jax.numpy as jnp
from jax.experimental import pallas as pl


def kernel(unique_node_ids, unique_messages, timestamps, memory, last_update, W, b):
    raise NotImplementedError("write your pallas kernel here")


# speedup vs baseline: 2.0875x; 2.0875x over previous
import jax
import jax.numpy as jnp
from jax import lax
from jax.experimental import pallas as pl
from jax.experimental.pallas import tpu as pltpu
from jax.experimental.pallas import tpu_sc as plsc
from jax._src.pallas import mpmd as _mpmd

N_NODES = 1_000_000
MEM_DIM = 64
MSG_DIM = 128
B = 65536

NUM_WORKERS = 32
CHUNK = B // NUM_WORKERS
NSUB = 4
SUB = CHUNK // NSUB

LU_COLS = 64
LU_ROWS = N_NODES // LU_COLS

COPY_ROWS = 8000
LU_BLOCK = LU_ROWS // (N_NODES // MEM_DIM // COPY_ROWS)


def _matmul_body(msg_ref, w_ref, b_ref, out_ref):
    out_ref[...] = (
        jnp.dot(msg_ref[...], w_ref[...], preferred_element_type=jnp.float32)
        + b_ref[...]
    )


def _copy_body(mem_ref, lu_ref, mem_out_ref, lu_out_ref):
    mem_out_ref[...] = mem_ref[...]
    lu_out_ref[...] = lu_ref[...]


def _sc_scatter_body(ids_hbm, lo_hbm, ts_hbm, upd_hbm, mem0_hbm, lu0_hbm,
                     out_mem_hbm, out_lu_hbm,
                     idx_v, lo_v, rows_v, tse_v, sem, sem2):
    wid = lax.axis_index("s") * 2 + lax.axis_index("c")
    pltpu.sync_copy(ids_hbm.at[wid], idx_v)
    pltpu.sync_copy(lo_hbm.at[wid], lo_v)
    for j in range(NSUB):
        g1 = pltpu.async_copy(upd_hbm.at[lo_v.at[j]], rows_v, sem)
        g2 = pltpu.async_copy(ts_hbm.at[lo_v.at[j]], tse_v, sem2)
        g1.wait()
        g2.wait()
        s1 = pltpu.async_copy(rows_v, out_mem_hbm.at[idx_v.at[j]], sem)
        s2 = pltpu.async_copy(tse_v, out_lu_hbm.at[idx_v.at[j]], sem2)
        s1.wait()
        s2.wait()


def _tc_matmul(messages, W, b):
    grid = B // CHUNK
    return pl.pallas_call(
        _matmul_body,
        grid=(grid,),
        in_specs=[
            pl.BlockSpec((CHUNK, MSG_DIM), lambda i: (i, 0)),
            pl.BlockSpec((MSG_DIM, MEM_DIM), lambda i: (0, 0)),
            pl.BlockSpec((1, MEM_DIM), lambda i: (0, 0)),
        ],
        out_specs=pl.BlockSpec((CHUNK, MEM_DIM), lambda i: (i, 0)),
        out_shape=jax.ShapeDtypeStruct((B, MEM_DIM), jnp.float32),
    )(messages, W, b.reshape(1, MEM_DIM))


def _tc_copy(memory, last_update):
    grid = memory.shape[0] // COPY_ROWS
    lu2d = last_update.reshape(LU_ROWS, LU_COLS)
    return pl.pallas_call(
        _copy_body,
        grid=(grid,),
        in_specs=[
            pl.BlockSpec((COPY_ROWS, MEM_DIM), lambda i: (i, 0)),
            pl.BlockSpec((LU_BLOCK, LU_COLS), lambda i: (i, 0)),
        ],
        out_specs=[
            pl.BlockSpec((COPY_ROWS, MEM_DIM), lambda i: (i, 0)),
            pl.BlockSpec((LU_BLOCK, LU_COLS), lambda i: (i, 0)),
        ],
        out_shape=[
            jax.ShapeDtypeStruct((memory.shape[0], MEM_DIM), jnp.float32),
            jax.ShapeDtypeStruct((LU_ROWS, LU_COLS), jnp.float32),
        ],
    )(memory, lu2d)


def _sc_scatter(ids3, lo3, ts, updated, mem0, lu0):
    mesh = plsc.VectorSubcoreMesh(core_axis_name="c", subcore_axis_name="s")
    fn = _mpmd._mpmd_map(
        [(mesh, _sc_scatter_body)],
        [
            jax.ShapeDtypeStruct((N_NODES, MEM_DIM), jnp.float32),
            jax.ShapeDtypeStruct((N_NODES,), jnp.float32),
        ],
        input_output_aliases={4: 0, 5: 1},
        compiler_params=pltpu.CompilerParams(use_tc_tiling_on_sc=False),
        scratch_types=[
            pltpu.VMEM((NSUB, SUB), jnp.int32),
            pltpu.VMEM((NSUB, SUB), jnp.int32),
            pltpu.VMEM((SUB, MEM_DIM), jnp.float32),
            pltpu.VMEM((SUB,), jnp.float32),
            pltpu.SemaphoreType.DMA,
            pltpu.SemaphoreType.DMA,
        ],
    )
    return fn(ids3, lo3, ts, updated, mem0, lu0)


def kernel(unique_node_ids, unique_messages, timestamps, memory, last_update,
           W, b):
    ids = unique_node_ids.astype(jnp.int32)
    last_occ = (jnp.searchsorted(ids, ids, side="right") - 1).astype(jnp.int32)
    ids3 = ids.reshape(NUM_WORKERS, NSUB, SUB)
    lo3 = last_occ.reshape(NUM_WORKERS, NSUB, SUB)
    updated = _tc_matmul(unique_messages, W, b)
    mem0, lu0 = _tc_copy(memory, last_update)
    out_mem, out_lu = _sc_scatter(ids3, lo3, timestamps, updated, mem0,
                                  lu0.reshape(N_NODES))
    return (out_mem, out_lu)


# speedup vs baseline: 5.4814x; 2.6258x over previous
import jax
import jax.numpy as jnp
from jax import lax
from jax.experimental import pallas as pl
from jax.experimental.pallas import tpu as pltpu
from jax.experimental.pallas import tpu_sc as plsc
from jax._src.pallas import mpmd as _mpmd

N_NODES = 1_000_000
MEM_DIM = 64
MSG_DIM = 128
B = 65536

NUM_WORKERS = 32
CHUNK = B // NUM_WORKERS
NSUB = 4
SUB = CHUNK // NSUB

def _matmul_body(msg_ref, w_ref, b_ref, out_ref):
    out_ref[...] = (
        jnp.dot(msg_ref[...], w_ref[...], preferred_element_type=jnp.float32)
        + b_ref[...]
    )


def _sc_scatter_body(ids_hbm, lo_hbm, ts_hbm, upd_hbm, mem0_hbm, lu0_hbm,
                     out_mem_hbm, out_lu_hbm,
                     idx_v, lo_v, rows_v, tse_v, sem, sem2):
    wid = lax.axis_index("s") * 2 + lax.axis_index("c")
    pltpu.sync_copy(ids_hbm.at[wid], idx_v)
    pltpu.sync_copy(lo_hbm.at[wid], lo_v)
    for j in range(NSUB):
        g1 = pltpu.async_copy(upd_hbm.at[lo_v.at[j]], rows_v, sem)
        g2 = pltpu.async_copy(ts_hbm.at[lo_v.at[j]], tse_v, sem2)
        g1.wait()
        g2.wait()
        s1 = pltpu.async_copy(rows_v, out_mem_hbm.at[idx_v.at[j]], sem)
        s2 = pltpu.async_copy(tse_v, out_lu_hbm.at[idx_v.at[j]], sem2)
        s1.wait()
        s2.wait()


def _tc_matmul(messages, W, b):
    grid = B // CHUNK
    return pl.pallas_call(
        _matmul_body,
        grid=(grid,),
        in_specs=[
            pl.BlockSpec((CHUNK, MSG_DIM), lambda i: (i, 0)),
            pl.BlockSpec((MSG_DIM, MEM_DIM), lambda i: (0, 0)),
            pl.BlockSpec((1, MEM_DIM), lambda i: (0, 0)),
        ],
        out_specs=pl.BlockSpec((CHUNK, MEM_DIM), lambda i: (i, 0)),
        out_shape=jax.ShapeDtypeStruct((B, MEM_DIM), jnp.float32),
    )(messages, W, b.reshape(1, MEM_DIM))


def _sc_scatter(ids3, lo3, ts, updated, mem0, lu0):
    mesh = plsc.VectorSubcoreMesh(core_axis_name="c", subcore_axis_name="s")
    fn = _mpmd._mpmd_map(
        [(mesh, _sc_scatter_body)],
        [
            jax.ShapeDtypeStruct((N_NODES, MEM_DIM), jnp.float32),
            jax.ShapeDtypeStruct((N_NODES,), jnp.float32),
        ],
        input_output_aliases={4: 0, 5: 1},
        compiler_params=pltpu.CompilerParams(use_tc_tiling_on_sc=False),
        scratch_types=[
            pltpu.VMEM((NSUB, SUB), jnp.int32),
            pltpu.VMEM((NSUB, SUB), jnp.int32),
            pltpu.VMEM((SUB, MEM_DIM), jnp.float32),
            pltpu.VMEM((SUB,), jnp.float32),
            pltpu.SemaphoreType.DMA,
            pltpu.SemaphoreType.DMA,
        ],
    )
    return fn(ids3, lo3, ts, updated, mem0, lu0)


def kernel(unique_node_ids, unique_messages, timestamps, memory, last_update,
           W, b):
    ids = unique_node_ids.astype(jnp.int32)
    iota = jnp.arange(B, dtype=jnp.int32)
    is_last = jnp.concatenate(
        [ids[1:] != ids[:-1], jnp.ones((1,), dtype=bool)])
    last_occ = lax.cummin(jnp.where(is_last, iota, B), axis=0, reverse=True)
    ids3 = ids.reshape(NUM_WORKERS, NSUB, SUB)
    lo3 = last_occ.reshape(NUM_WORKERS, NSUB, SUB)
    updated = _tc_matmul(unique_messages, W, b)
    out_mem, out_lu = _sc_scatter(ids3, lo3, timestamps, updated, memory,
                                  last_update)
    return (out_mem, out_lu)
